# baseline (speedup 1.0000x reference)
import functools

import jax
import jax.numpy as jnp
from jax import lax
from jax.experimental import pallas as pl
from jax.experimental.pallas import tpu as pltpu
from jax.experimental.pallas import tpu_sc as plsc

N = 100000
D = 128
G = 256
EPS = 1e-5

R = 800
NBLK = N // R
KSEG = 2
CSZ = 200
NCHK = R // CSZ


Q = 4
QR = R // Q


def _issue_x(x_hbm, xbuf, xsem, blk):
    slot = blk % 2
    for q in range(Q):
        pltpu.make_async_copy(
            x_hbm.at[pl.ds(blk * R + q * QR, QR), :],
            xbuf.at[slot, pl.ds(q * QR, QR), :],
            xsem.at[slot, q]).start()


def _wait_x(x_hbm, xbuf, xsem, blk):
    slot = blk % 2
    for q in range(Q):
        pltpu.make_async_copy(
            x_hbm.at[pl.ds(blk * R + q * QR, QR), :],
            xbuf.at[slot, pl.ds(q * QR, QR), :],
            xsem.at[slot, q]).wait()


def _tc_body(firsts_ref, lasts_ref, x_hbm, w_ref, b_ref, gamma_ref, beta_ref,
             batch_ref, pooled_ref, xbuf, pool_acc, h_scr, sum_acc, sq_acc,
             xsem):
    i = pl.program_id(0)

    @pl.when(i == 0)
    def _init():
        pool_acc[...] = jnp.full((G, D), -jnp.inf, jnp.float32)
        sum_acc[...] = jnp.zeros((8, D), jnp.float32)
        sq_acc[...] = jnp.zeros((8, D), jnp.float32)
        _issue_x(x_hbm, xbuf, xsem, 0)

    @pl.when(i + 1 < NBLK)
    def _prefetch():
        _issue_x(x_hbm, xbuf, xsem, i + 1)

    _wait_x(x_hbm, xbuf, xsem, i)
    slot = i % 2
    w = w_ref[...]
    bias = b_ref[...]
    s0 = firsts_ref[i]
    s1 = lasts_ref[i]

    rs = jnp.zeros((8, D), jnp.float32)
    rq = jnp.zeros((8, D), jnp.float32)
    cm = [jnp.full((1, D), -jnp.inf, jnp.float32) for _ in range(KSEG)]
    for c in range(NCHK):
        xc = xbuf[slot, pl.ds(c * CSZ, CSZ), :]
        hc = jnp.dot(xc, w, preferred_element_type=jnp.float32) + bias
        h_scr[pl.ds(c * CSZ, CSZ), :] = hc
        rs = rs + jnp.sum(hc.reshape(CSZ // 8, 8, D), axis=0)
        rq = rq + jnp.sum((hc * hc).reshape(CSZ // 8, 8, D), axis=0)
        bc = batch_ref[pl.ds(c * CSZ, CSZ), :]
        for k in range(KSEG):
            cmk = jnp.max(jnp.where(bc == s0 + k, hc, -jnp.inf), axis=0,
                          keepdims=True)
            cm[k] = jnp.maximum(cm[k], cmk)
    sum_acc[...] += rs
    sq_acc[...] += rq
    for k in range(KSEG):
        idx = jnp.minimum(s0 + k, G - 1)
        cur = pool_acc[pl.ds(idx, 1), :]
        pool_acc[pl.ds(idx, 1), :] = jnp.maximum(cur, cm[k])

    bcol = batch_ref[...]
    def seg_body(seg, carry):
        colmax = jnp.max(jnp.where(bcol == seg, h_scr[...], -jnp.inf), axis=0,
                         keepdims=True)
        cur = pool_acc[pl.ds(seg, 1), :]
        pool_acc[pl.ds(seg, 1), :] = jnp.maximum(cur, colmax)
        return carry

    lax.fori_loop(s0 + KSEG, s1 + 1, seg_body, 0)

    @pl.when(i == NBLK - 1)
    def _finalize():
        tot = jnp.sum(sum_acc[...], axis=0, keepdims=True)
        tot2 = jnp.sum(sq_acc[...], axis=0, keepdims=True)
        mean = tot * (1.0 / N)
        var = tot2 * (1.0 / N) - mean * mean
        scale = gamma_ref[...] * lax.rsqrt(var + EPS)
        shift = beta_ref[...] - mean * scale
        pooled_ref[...] = jnp.maximum(pool_acc[...] * scale + shift, 0.0)


def _tc_pass(x, batch_col, firsts, lasts, W, b, gamma, beta):
    return pl.pallas_call(
        _tc_body,
        grid=(NBLK,),
        in_specs=[
            pl.BlockSpec(memory_space=pltpu.SMEM),
            pl.BlockSpec(memory_space=pltpu.SMEM),
            pl.BlockSpec(memory_space=pltpu.MemorySpace.HBM),
            pl.BlockSpec((D, D), lambda i: (0, 0)),
            pl.BlockSpec((1, D), lambda i: (0, 0)),
            pl.BlockSpec((1, D), lambda i: (0, 0)),
            pl.BlockSpec((1, D), lambda i: (0, 0)),
            pl.BlockSpec((R, 1), lambda i: (i, 0)),
        ],
        out_specs=pl.BlockSpec((G, D), lambda i: (0, 0)),
        out_shape=jax.ShapeDtypeStruct((G, D), jnp.float32),
        scratch_shapes=[
            pltpu.VMEM((2, R, D), jnp.float32),
            pltpu.VMEM((G, D), jnp.float32),
            pltpu.VMEM((R, D), jnp.float32),
            pltpu.VMEM((8, D), jnp.float32),
            pltpu.VMEM((8, D), jnp.float32),
            pltpu.SemaphoreType.DMA((2, Q)),
        ],
        compiler_params=pltpu.CompilerParams(
            dimension_semantics=("arbitrary",),
        ),
    )(firsts, lasts, x, W, b.reshape(1, D), gamma.reshape(1, D),
      beta.reshape(1, D), batch_col)


CH = 128
NW = 32
NCH = 25
WROWS = NCH * CH
IB = 6
_MAXOFF = N - CH
_MAXBASE = N - WROWS


def _sc_expand(pooled, batch):
    mesh = plsc.VectorSubcoreMesh(core_axis_name="c", subcore_axis_name="s")

    @functools.partial(
        pl.kernel,
        mesh=mesh,
        out_type=jax.ShapeDtypeStruct((N, D), jnp.float32),
        scratch_types=[
            pltpu.VMEM((WROWS,), jnp.int32),
            pltpu.VMEM((IB, CH, D), jnp.float32),
            pltpu.VMEM_SHARED((G, D), jnp.float32),
            pltpu.SemaphoreType.DMA,
            pltpu.SemaphoreType.DMA,
        ],
    )
    def expand(pooled_hbm, batch_hbm, out_hbm, idx_all, row_bufs, pooled_sh,
               sem_g, sem_w):
        c = lax.axis_index("c")
        s = lax.axis_index("s")
        wid = s * 2 + c
        base = wid * WROWS
        lbase = pl.multiple_of(jnp.minimum(base, _MAXBASE), 8)
        @pl.when(s == 0)
        def _stage():
            pltpu.sync_copy(pooled_hbm, pooled_sh)
        plsc.subcore_barrier()
        pltpu.sync_copy(batch_hbm.at[pl.ds(lbase, WROWS)], idx_all)

        offs = [pl.multiple_of(jnp.minimum(base + j * CH, _MAXOFF), 8)
                for j in range(NCH)]
        loffs = [pl.multiple_of(offs[j] - lbase, 8) for j in range(NCH)]

        gh = [None] * NCH
        wh = [None] * NCH

        def gather(j):
            return pltpu.async_copy(
                pooled_sh.at[idx_all.at[pl.ds(loffs[j], CH)]],
                row_bufs.at[j % IB], sem_g)

        def write(j):
            return pltpu.async_copy(
                row_bufs.at[j % IB], out_hbm.at[pl.ds(offs[j], CH)], sem_w)

        for k in range(IB - 1):
            gh[k] = gather(k)
        for j in range(NCH):
            nxt = j + IB - 1
            if nxt < NCH:
                if nxt - IB >= 0:
                    wh[nxt - IB].wait()
                gh[nxt] = gather(nxt)
            gh[j].wait()
            wh[j] = write(j)
        for j in range(max(0, NCH - IB), NCH):
            wh[j].wait()

    return expand(pooled, batch)


def kernel(x, stroke_idx, batch, W, b, gamma, beta):
    del stroke_idx
    batch = batch.astype(jnp.int32)
    batch_col = batch.reshape(N, 1)
    firsts = batch[::R]
    lasts = batch[R - 1::R]
    pooled = _tc_pass(x, batch_col, firsts, lasts, W, b, gamma, beta)
    return _sc_expand(pooled, batch)


# speedup vs baseline: 1.0188x; 1.0188x over previous
import functools

import jax
import jax.numpy as jnp
from jax import lax
from jax.experimental import pallas as pl
from jax.experimental.pallas import tpu as pltpu
from jax.experimental.pallas import tpu_sc as plsc

N = 100000
D = 128
G = 256
EPS = 1e-5

R = 800
NBLK = N // R
KSEG = 3
CSZ = 200
NCHK = R // CSZ


Q = 4
QR = R // Q


def _issue_x(x_hbm, xbuf, xsem, blk):
    slot = blk % 2
    for q in range(Q):
        pltpu.make_async_copy(
            x_hbm.at[pl.ds(blk * R + q * QR, QR), :],
            xbuf.at[slot, pl.ds(q * QR, QR), :],
            xsem.at[slot, q]).start()


def _wait_x(x_hbm, xbuf, xsem, blk):
    slot = blk % 2
    for q in range(Q):
        pltpu.make_async_copy(
            x_hbm.at[pl.ds(blk * R + q * QR, QR), :],
            xbuf.at[slot, pl.ds(q * QR, QR), :],
            xsem.at[slot, q]).wait()


def _tc_body(firsts_ref, lasts_ref, x_hbm, w_ref, b_ref, gamma_ref, beta_ref,
             batch_ref, pooled_ref, xbuf, pool_acc, h_scr, sum_acc, sq_acc,
             xsem):
    i = pl.program_id(0)

    @pl.when(i == 0)
    def _init():
        pool_acc[...] = jnp.full((G, D), -jnp.inf, jnp.float32)
        sum_acc[...] = jnp.zeros((8, D), jnp.float32)
        sq_acc[...] = jnp.zeros((8, D), jnp.float32)
        _issue_x(x_hbm, xbuf, xsem, 0)

    @pl.when(i + 1 < NBLK)
    def _prefetch():
        _issue_x(x_hbm, xbuf, xsem, i + 1)

    _wait_x(x_hbm, xbuf, xsem, i)
    slot = i % 2
    w = w_ref[...]
    bias = b_ref[...]
    s0 = firsts_ref[i]
    s1 = lasts_ref[i]

    rs = jnp.zeros((8, D), jnp.float32)
    rq = jnp.zeros((8, D), jnp.float32)
    cm = [jnp.full((1, D), -jnp.inf, jnp.float32) for _ in range(KSEG)]
    for c in range(NCHK):
        xc = xbuf[slot, pl.ds(c * CSZ, CSZ), :]
        hc = jnp.dot(xc, w, preferred_element_type=jnp.float32) + bias
        h_scr[pl.ds(c * CSZ, CSZ), :] = hc
        rs = rs + jnp.sum(hc.reshape(CSZ // 8, 8, D), axis=0)
        rq = rq + jnp.sum((hc * hc).reshape(CSZ // 8, 8, D), axis=0)
        bc = batch_ref[pl.ds(c * CSZ, CSZ), :]
        for k in range(KSEG):
            cmk = jnp.max(jnp.where(bc == s0 + k, hc, -jnp.inf), axis=0,
                          keepdims=True)
            cm[k] = jnp.maximum(cm[k], cmk)
    sum_acc[...] += rs
    sq_acc[...] += rq
    for k in range(KSEG):
        idx = jnp.minimum(s0 + k, G - 1)
        cur = pool_acc[pl.ds(idx, 1), :]
        pool_acc[pl.ds(idx, 1), :] = jnp.maximum(cur, cm[k])

    bcol = batch_ref[...]
    def seg_body(seg, carry):
        colmax = jnp.max(jnp.where(bcol == seg, h_scr[...], -jnp.inf), axis=0,
                         keepdims=True)
        cur = pool_acc[pl.ds(seg, 1), :]
        pool_acc[pl.ds(seg, 1), :] = jnp.maximum(cur, colmax)
        return carry

    lax.fori_loop(s0 + KSEG, s1 + 1, seg_body, 0)

    @pl.when(i == NBLK - 1)
    def _finalize():
        tot = jnp.sum(sum_acc[...], axis=0, keepdims=True)
        tot2 = jnp.sum(sq_acc[...], axis=0, keepdims=True)
        mean = tot * (1.0 / N)
        var = tot2 * (1.0 / N) - mean * mean
        scale = gamma_ref[...] * lax.rsqrt(var + EPS)
        shift = beta_ref[...] - mean * scale
        pooled_ref[...] = jnp.maximum(pool_acc[...] * scale + shift, 0.0)


def _tc_pass(x, batch_col, firsts, lasts, W, b, gamma, beta):
    return pl.pallas_call(
        _tc_body,
        grid=(NBLK,),
        in_specs=[
            pl.BlockSpec(memory_space=pltpu.SMEM),
            pl.BlockSpec(memory_space=pltpu.SMEM),
            pl.BlockSpec(memory_space=pltpu.MemorySpace.HBM),
            pl.BlockSpec((D, D), lambda i: (0, 0)),
            pl.BlockSpec((1, D), lambda i: (0, 0)),
            pl.BlockSpec((1, D), lambda i: (0, 0)),
            pl.BlockSpec((1, D), lambda i: (0, 0)),
            pl.BlockSpec((R, 1), lambda i: (i, 0)),
        ],
        out_specs=pl.BlockSpec((G, D), lambda i: (0, 0)),
        out_shape=jax.ShapeDtypeStruct((G, D), jnp.float32),
        scratch_shapes=[
            pltpu.VMEM((2, R, D), jnp.float32),
            pltpu.VMEM((G, D), jnp.float32),
            pltpu.VMEM((R, D), jnp.float32),
            pltpu.VMEM((8, D), jnp.float32),
            pltpu.VMEM((8, D), jnp.float32),
            pltpu.SemaphoreType.DMA((2, Q)),
        ],
        compiler_params=pltpu.CompilerParams(
            dimension_semantics=("arbitrary",),
        ),
    )(firsts, lasts, x, W, b.reshape(1, D), gamma.reshape(1, D),
      beta.reshape(1, D), batch_col)


CH = 128
NW = 32
NCH = 25
WROWS = NCH * CH
IB = 6
_MAXOFF = N - CH
_MAXBASE = N - WROWS


def _sc_expand(pooled, batch):
    mesh = plsc.VectorSubcoreMesh(core_axis_name="c", subcore_axis_name="s")

    @functools.partial(
        pl.kernel,
        mesh=mesh,
        out_type=jax.ShapeDtypeStruct((N, D), jnp.float32),
        scratch_types=[
            pltpu.VMEM((WROWS,), jnp.int32),
            pltpu.VMEM((IB, CH, D), jnp.float32),
            pltpu.VMEM_SHARED((G, D), jnp.float32),
            pltpu.SemaphoreType.DMA,
            pltpu.SemaphoreType.DMA,
        ],
    )
    def expand(pooled_hbm, batch_hbm, out_hbm, idx_all, row_bufs, pooled_sh,
               sem_g, sem_w):
        c = lax.axis_index("c")
        s = lax.axis_index("s")
        wid = s * 2 + c
        base = wid * WROWS
        lbase = pl.multiple_of(jnp.minimum(base, _MAXBASE), 8)
        @pl.when(s == 0)
        def _stage():
            pltpu.sync_copy(pooled_hbm, pooled_sh)
        plsc.subcore_barrier()
        pltpu.sync_copy(batch_hbm.at[pl.ds(lbase, WROWS)], idx_all)

        offs = [pl.multiple_of(jnp.minimum(base + j * CH, _MAXOFF), 8)
                for j in range(NCH)]
        loffs = [pl.multiple_of(offs[j] - lbase, 8) for j in range(NCH)]

        gh = [None] * NCH
        wh = [None] * NCH

        def gather(j):
            return pltpu.async_copy(
                pooled_sh.at[idx_all.at[pl.ds(loffs[j], CH)]],
                row_bufs.at[j % IB], sem_g)

        def write(j):
            return pltpu.async_copy(
                row_bufs.at[j % IB], out_hbm.at[pl.ds(offs[j], CH)], sem_w)

        for k in range(IB - 1):
            gh[k] = gather(k)
        for j in range(NCH):
            nxt = j + IB - 1
            if nxt < NCH:
                if nxt - IB >= 0:
                    wh[nxt - IB].wait()
                gh[nxt] = gather(nxt)
            gh[j].wait()
            wh[j] = write(j)
        for j in range(max(0, NCH - IB), NCH):
            wh[j].wait()

    return expand(pooled, batch)


def kernel(x, stroke_idx, batch, W, b, gamma, beta):
    del stroke_idx
    batch = batch.astype(jnp.int32)
    batch_col = batch.reshape(N, 1)
    firsts = batch[::R]
    lasts = batch[R - 1::R]
    pooled = _tc_pass(x, batch_col, firsts, lasts, W, b, gamma, beta)
    return _sc_expand(pooled, batch)


# speedup vs baseline: 1.0706x; 1.0508x over previous
import functools

import jax
import jax.numpy as jnp
from jax import lax
from jax.experimental import pallas as pl
from jax.experimental.pallas import tpu as pltpu
from jax.experimental.pallas import tpu_sc as plsc

N = 100000
D = 128
G = 256
EPS = 1e-5

R = 1000
NBLK = N // R
KSEG = 3
CSZ = 200
NCHK = R // CSZ


Q = 4
QR = R // Q


def _issue_x(x_hbm, xbuf, xsem, blk):
    slot = blk % 2
    for q in range(Q):
        pltpu.make_async_copy(
            x_hbm.at[pl.ds(blk * R + q * QR, QR), :],
            xbuf.at[slot, pl.ds(q * QR, QR), :],
            xsem.at[slot, q]).start()


def _wait_x(x_hbm, xbuf, xsem, blk):
    slot = blk % 2
    for q in range(Q):
        pltpu.make_async_copy(
            x_hbm.at[pl.ds(blk * R + q * QR, QR), :],
            xbuf.at[slot, pl.ds(q * QR, QR), :],
            xsem.at[slot, q]).wait()


def _tc_body(firsts_ref, lasts_ref, x_hbm, w_ref, b_ref, gamma_ref, beta_ref,
             batch_ref, pooled_ref, xbuf, pool_acc, h_scr, sum_acc, sq_acc,
             xsem):
    i = pl.program_id(0)

    @pl.when(i == 0)
    def _init():
        pool_acc[...] = jnp.full((G, D), -jnp.inf, jnp.float32)
        sum_acc[...] = jnp.zeros((8, D), jnp.float32)
        sq_acc[...] = jnp.zeros((8, D), jnp.float32)
        _issue_x(x_hbm, xbuf, xsem, 0)

    @pl.when(i + 1 < NBLK)
    def _prefetch():
        _issue_x(x_hbm, xbuf, xsem, i + 1)

    _wait_x(x_hbm, xbuf, xsem, i)
    slot = i % 2
    w = w_ref[...]
    bias = b_ref[...]
    s0 = firsts_ref[i]
    s1 = lasts_ref[i]

    rs = jnp.zeros((8, D), jnp.float32)
    rq = jnp.zeros((8, D), jnp.float32)
    cm = [jnp.full((1, D), -jnp.inf, jnp.float32) for _ in range(KSEG)]
    for c in range(NCHK):
        xc = xbuf[slot, pl.ds(c * CSZ, CSZ), :]
        hc = jnp.dot(xc, w, preferred_element_type=jnp.float32) + bias
        h_scr[pl.ds(c * CSZ, CSZ), :] = hc
        rs = rs + jnp.sum(hc.reshape(CSZ // 8, 8, D), axis=0)
        rq = rq + jnp.sum((hc * hc).reshape(CSZ // 8, 8, D), axis=0)
        bc = batch_ref[pl.ds(c * CSZ, CSZ), :]
        for k in range(KSEG):
            cmk = jnp.max(jnp.where(bc == s0 + k, hc, -jnp.inf), axis=0,
                          keepdims=True)
            cm[k] = jnp.maximum(cm[k], cmk)
    sum_acc[...] += rs
    sq_acc[...] += rq
    for k in range(KSEG):
        idx = jnp.minimum(s0 + k, G - 1)
        cur = pool_acc[pl.ds(idx, 1), :]
        pool_acc[pl.ds(idx, 1), :] = jnp.maximum(cur, cm[k])

    bcol = batch_ref[...]
    def seg_body(seg, carry):
        colmax = jnp.max(jnp.where(bcol == seg, h_scr[...], -jnp.inf), axis=0,
                         keepdims=True)
        cur = pool_acc[pl.ds(seg, 1), :]
        pool_acc[pl.ds(seg, 1), :] = jnp.maximum(cur, colmax)
        return carry

    lax.fori_loop(s0 + KSEG, s1 + 1, seg_body, 0)

    @pl.when(i == NBLK - 1)
    def _finalize():
        tot = jnp.sum(sum_acc[...], axis=0, keepdims=True)
        tot2 = jnp.sum(sq_acc[...], axis=0, keepdims=True)
        mean = tot * (1.0 / N)
        var = tot2 * (1.0 / N) - mean * mean
        scale = gamma_ref[...] * lax.rsqrt(var + EPS)
        shift = beta_ref[...] - mean * scale
        pooled_ref[...] = jnp.maximum(pool_acc[...] * scale + shift, 0.0)


def _tc_pass(x, batch_col, firsts, lasts, W, b, gamma, beta):
    return pl.pallas_call(
        _tc_body,
        grid=(NBLK,),
        in_specs=[
            pl.BlockSpec(memory_space=pltpu.SMEM),
            pl.BlockSpec(memory_space=pltpu.SMEM),
            pl.BlockSpec(memory_space=pltpu.MemorySpace.HBM),
            pl.BlockSpec((D, D), lambda i: (0, 0)),
            pl.BlockSpec((1, D), lambda i: (0, 0)),
            pl.BlockSpec((1, D), lambda i: (0, 0)),
            pl.BlockSpec((1, D), lambda i: (0, 0)),
            pl.BlockSpec((R, 1), lambda i: (i, 0)),
        ],
        out_specs=pl.BlockSpec((G, D), lambda i: (0, 0)),
        out_shape=jax.ShapeDtypeStruct((G, D), jnp.float32),
        scratch_shapes=[
            pltpu.VMEM((2, R, D), jnp.float32),
            pltpu.VMEM((G, D), jnp.float32),
            pltpu.VMEM((R, D), jnp.float32),
            pltpu.VMEM((8, D), jnp.float32),
            pltpu.VMEM((8, D), jnp.float32),
            pltpu.SemaphoreType.DMA((2, Q)),
        ],
        compiler_params=pltpu.CompilerParams(
            dimension_semantics=("arbitrary",),
        ),
    )(firsts, lasts, x, W, b.reshape(1, D), gamma.reshape(1, D),
      beta.reshape(1, D), batch_col)


CH = 128
NW = 32
NCH = 25
WROWS = NCH * CH
IB = 6
_MAXOFF = N - CH
_MAXBASE = N - WROWS


def _sc_expand(pooled, batch):
    mesh = plsc.VectorSubcoreMesh(core_axis_name="c", subcore_axis_name="s")

    @functools.partial(
        pl.kernel,
        mesh=mesh,
        out_type=jax.ShapeDtypeStruct((N, D), jnp.float32),
        scratch_types=[
            pltpu.VMEM((WROWS,), jnp.int32),
            pltpu.VMEM((IB, CH, D), jnp.float32),
            pltpu.VMEM_SHARED((G, D), jnp.float32),
            pltpu.SemaphoreType.DMA,
            pltpu.SemaphoreType.DMA,
        ],
    )
    def expand(pooled_hbm, batch_hbm, out_hbm, idx_all, row_bufs, pooled_sh,
               sem_g, sem_w):
        c = lax.axis_index("c")
        s = lax.axis_index("s")
        wid = s * 2 + c
        base = wid * WROWS
        lbase = pl.multiple_of(jnp.minimum(base, _MAXBASE), 8)
        @pl.when(s == 0)
        def _stage():
            pltpu.sync_copy(pooled_hbm, pooled_sh)
        plsc.subcore_barrier()
        pltpu.sync_copy(batch_hbm.at[pl.ds(lbase, WROWS)], idx_all)

        offs = [pl.multiple_of(jnp.minimum(base + j * CH, _MAXOFF), 8)
                for j in range(NCH)]
        loffs = [pl.multiple_of(offs[j] - lbase, 8) for j in range(NCH)]

        gh = [None] * NCH
        wh = [None] * NCH

        def gather(j):
            return pltpu.async_copy(
                pooled_sh.at[idx_all.at[pl.ds(loffs[j], CH)]],
                row_bufs.at[j % IB], sem_g)

        def write(j):
            return pltpu.async_copy(
                row_bufs.at[j % IB], out_hbm.at[pl.ds(offs[j], CH)], sem_w)

        for k in range(IB - 1):
            gh[k] = gather(k)
        for j in range(NCH):
            nxt = j + IB - 1
            if nxt < NCH:
                if nxt - IB >= 0:
                    wh[nxt - IB].wait()
                gh[nxt] = gather(nxt)
            gh[j].wait()
            wh[j] = write(j)
        for j in range(max(0, NCH - IB), NCH):
            wh[j].wait()

    return expand(pooled, batch)


def kernel(x, stroke_idx, batch, W, b, gamma, beta):
    del stroke_idx
    batch = batch.astype(jnp.int32)
    batch_col = batch.reshape(N, 1)
    firsts = batch[::R]
    lasts = batch[R - 1::R]
    pooled = _tc_pass(x, batch_col, firsts, lasts, W, b, gamma, beta)
    return _sc_expand(pooled, batch)


# speedup vs baseline: 1.0787x; 1.0076x over previous
import functools

import jax
import jax.numpy as jnp
from jax import lax
from jax.experimental import pallas as pl
from jax.experimental.pallas import tpu as pltpu
from jax.experimental.pallas import tpu_sc as plsc

N = 100000
D = 128
G = 256
EPS = 1e-5

R = 1000
NBLK = N // R
KSEG = 4
CSZ = 200
NCHK = R // CSZ


Q = 4
QR = R // Q


def _issue_x(x_hbm, xbuf, xsem, blk):
    slot = blk % 2
    for q in range(Q):
        pltpu.make_async_copy(
            x_hbm.at[pl.ds(blk * R + q * QR, QR), :],
            xbuf.at[slot, pl.ds(q * QR, QR), :],
            xsem.at[slot, q]).start()


def _wait_x(x_hbm, xbuf, xsem, blk):
    slot = blk % 2
    for q in range(Q):
        pltpu.make_async_copy(
            x_hbm.at[pl.ds(blk * R + q * QR, QR), :],
            xbuf.at[slot, pl.ds(q * QR, QR), :],
            xsem.at[slot, q]).wait()


def _tc_body(firsts_ref, lasts_ref, x_hbm, w_ref, b_ref, gamma_ref, beta_ref,
             batch_ref, pooled_ref, xbuf, pool_acc, h_scr, sum_acc, sq_acc,
             xsem):
    i = pl.program_id(0)

    @pl.when(i == 0)
    def _init():
        pool_acc[...] = jnp.full((G, D), -jnp.inf, jnp.float32)
        sum_acc[...] = jnp.zeros((8, D), jnp.float32)
        sq_acc[...] = jnp.zeros((8, D), jnp.float32)
        _issue_x(x_hbm, xbuf, xsem, 0)

    @pl.when(i + 1 < NBLK)
    def _prefetch():
        _issue_x(x_hbm, xbuf, xsem, i + 1)

    _wait_x(x_hbm, xbuf, xsem, i)
    slot = i % 2
    w = w_ref[...]
    bias = b_ref[...]
    s0 = firsts_ref[i]
    s1 = lasts_ref[i]

    rs = jnp.zeros((8, D), jnp.float32)
    rq = jnp.zeros((8, D), jnp.float32)
    cm = [jnp.full((1, D), -jnp.inf, jnp.float32) for _ in range(KSEG)]
    for c in range(NCHK):
        xc = xbuf[slot, pl.ds(c * CSZ, CSZ), :]
        hc = jnp.dot(xc, w, preferred_element_type=jnp.float32) + bias
        h_scr[pl.ds(c * CSZ, CSZ), :] = hc
        rs = rs + jnp.sum(hc.reshape(CSZ // 8, 8, D), axis=0)
        rq = rq + jnp.sum((hc * hc).reshape(CSZ // 8, 8, D), axis=0)
        bc = batch_ref[pl.ds(c * CSZ, CSZ), :]
        for k in range(KSEG):
            cmk = jnp.max(jnp.where(bc == s0 + k, hc, -jnp.inf), axis=0,
                          keepdims=True)
            cm[k] = jnp.maximum(cm[k], cmk)
    sum_acc[...] += rs
    sq_acc[...] += rq
    for k in range(KSEG):
        idx = jnp.minimum(s0 + k, G - 1)
        cur = pool_acc[pl.ds(idx, 1), :]
        pool_acc[pl.ds(idx, 1), :] = jnp.maximum(cur, cm[k])

    bcol = batch_ref[...]
    def seg_body(seg, carry):
        colmax = jnp.max(jnp.where(bcol == seg, h_scr[...], -jnp.inf), axis=0,
                         keepdims=True)
        cur = pool_acc[pl.ds(seg, 1), :]
        pool_acc[pl.ds(seg, 1), :] = jnp.maximum(cur, colmax)
        return carry

    lax.fori_loop(s0 + KSEG, s1 + 1, seg_body, 0)

    @pl.when(i == NBLK - 1)
    def _finalize():
        tot = jnp.sum(sum_acc[...], axis=0, keepdims=True)
        tot2 = jnp.sum(sq_acc[...], axis=0, keepdims=True)
        mean = tot * (1.0 / N)
        var = tot2 * (1.0 / N) - mean * mean
        scale = gamma_ref[...] * lax.rsqrt(var + EPS)
        shift = beta_ref[...] - mean * scale
        pooled_ref[...] = jnp.maximum(pool_acc[...] * scale + shift, 0.0)


def _tc_pass(x, batch_col, firsts, lasts, W, b, gamma, beta):
    return pl.pallas_call(
        _tc_body,
        grid=(NBLK,),
        in_specs=[
            pl.BlockSpec(memory_space=pltpu.SMEM),
            pl.BlockSpec(memory_space=pltpu.SMEM),
            pl.BlockSpec(memory_space=pltpu.MemorySpace.HBM),
            pl.BlockSpec((D, D), lambda i: (0, 0)),
            pl.BlockSpec((1, D), lambda i: (0, 0)),
            pl.BlockSpec((1, D), lambda i: (0, 0)),
            pl.BlockSpec((1, D), lambda i: (0, 0)),
            pl.BlockSpec((R, 1), lambda i: (i, 0)),
        ],
        out_specs=pl.BlockSpec((G, D), lambda i: (0, 0)),
        out_shape=jax.ShapeDtypeStruct((G, D), jnp.float32),
        scratch_shapes=[
            pltpu.VMEM((2, R, D), jnp.float32),
            pltpu.VMEM((G, D), jnp.float32),
            pltpu.VMEM((R, D), jnp.float32),
            pltpu.VMEM((8, D), jnp.float32),
            pltpu.VMEM((8, D), jnp.float32),
            pltpu.SemaphoreType.DMA((2, Q)),
        ],
        compiler_params=pltpu.CompilerParams(
            dimension_semantics=("arbitrary",),
        ),
    )(firsts, lasts, x, W, b.reshape(1, D), gamma.reshape(1, D),
      beta.reshape(1, D), batch_col)


CH = 128
NW = 32
NCH = 25
WROWS = NCH * CH
IB = 6
_MAXOFF = N - CH
_MAXBASE = N - WROWS


def _sc_expand(pooled, batch):
    mesh = plsc.VectorSubcoreMesh(core_axis_name="c", subcore_axis_name="s")

    @functools.partial(
        pl.kernel,
        mesh=mesh,
        out_type=jax.ShapeDtypeStruct((N, D), jnp.float32),
        scratch_types=[
            pltpu.VMEM((WROWS,), jnp.int32),
            pltpu.VMEM((IB, CH, D), jnp.float32),
            pltpu.VMEM_SHARED((G, D), jnp.float32),
            pltpu.SemaphoreType.DMA,
            pltpu.SemaphoreType.DMA,
        ],
    )
    def expand(pooled_hbm, batch_hbm, out_hbm, idx_all, row_bufs, pooled_sh,
               sem_g, sem_w):
        c = lax.axis_index("c")
        s = lax.axis_index("s")
        wid = s * 2 + c
        base = wid * WROWS
        lbase = pl.multiple_of(jnp.minimum(base, _MAXBASE), 8)
        @pl.when(s == 0)
        def _stage():
            pltpu.sync_copy(pooled_hbm, pooled_sh)
        plsc.subcore_barrier()
        pltpu.sync_copy(batch_hbm.at[pl.ds(lbase, WROWS)], idx_all)

        offs = [pl.multiple_of(jnp.minimum(base + j * CH, _MAXOFF), 8)
                for j in range(NCH)]
        loffs = [pl.multiple_of(offs[j] - lbase, 8) for j in range(NCH)]

        gh = [None] * NCH
        wh = [None] * NCH

        def gather(j):
            return pltpu.async_copy(
                pooled_sh.at[idx_all.at[pl.ds(loffs[j], CH)]],
                row_bufs.at[j % IB], sem_g)

        def write(j):
            return pltpu.async_copy(
                row_bufs.at[j % IB], out_hbm.at[pl.ds(offs[j], CH)], sem_w)

        for k in range(IB - 1):
            gh[k] = gather(k)
        for j in range(NCH):
            nxt = j + IB - 1
            if nxt < NCH:
                if nxt - IB >= 0:
                    wh[nxt - IB].wait()
                gh[nxt] = gather(nxt)
            gh[j].wait()
            wh[j] = write(j)
        for j in range(max(0, NCH - IB), NCH):
            wh[j].wait()

    return expand(pooled, batch)


def kernel(x, stroke_idx, batch, W, b, gamma, beta):
    del stroke_idx
    batch = batch.astype(jnp.int32)
    batch_col = batch.reshape(N, 1)
    firsts = batch[::R]
    lasts = batch[R - 1::R]
    pooled = _tc_pass(x, batch_col, firsts, lasts, W, b, gamma, beta)
    return _sc_expand(pooled, batch)
